# baseline (speedup 1.0000x reference)
import functools

import jax
import jax.numpy as jnp
from jax import lax
from jax.experimental import pallas as pl
from jax.experimental.pallas import tpu as pltpu
from jax.experimental.pallas import tpu_sc as plsc

_NC = 2
_NS = 16
_NW = _NC * _NS
_BB = 128
_SEG = 16


def _matmul_body(x_ref, w_ref, y_ref):
    y_ref[0] = jnp.dot(x_ref[...], w_ref[0], preferred_element_type=jnp.float32)


def _combine_body(p_ref, b_ref, o_ref):
    o_ref[...] = p_ref[0] + p_ref[1] + b_ref[...]


def _make_sc_scatter(n_acc, cout, bpt):
    rpt = n_acc // _NS
    nseg = bpt // _SEG

    mesh = plsc.VectorSubcoreMesh(
        core_axis_name="c", subcore_axis_name="s",
        num_cores=_NC, num_subcores=_NS)

    @functools.partial(
        pl.kernel,
        out_type=jax.ShapeDtypeStruct((_NC, n_acc, cout), jnp.float32),
        mesh=mesh,
        scratch_types=[
            pltpu.VMEM((_SEG, _BB), jnp.int32),
            pltpu.VMEM((_SEG, _BB), jnp.int32),
            pltpu.VMEM((2, _BB, cout), jnp.float32),
            pltpu.VMEM_SHARED((n_acc, cout), jnp.float32),
            pltpu.SemaphoreType.DMA,
            pltpu.SemaphoreType.DMA,
            pltpu.SemaphoreType.DMA,
            pltpu.SemaphoreType.DMA,
        ],
    )
    def sc_scatter(y_hbm, gidx_hbm, omap_hbm, out_hbm,
                   gidx_v, omap_v, rows_v, acc_sh, gs0, gs1, ss0, ss1):
        cid = lax.axis_index("c")
        sid = lax.axis_index("s")
        wid = cid * _NS + sid

        zero = jnp.zeros((16,), jnp.float32)

        @pl.loop(0, _BB)
        def _zero_rows(r):
            for q in range(cout // 16):
                rows_v[0, r, pl.ds(q * 16, 16)] = zero

        for t in range(rpt // _BB):
            pltpu.sync_copy(rows_v.at[0],
                            acc_sh.at[pl.ds(sid * rpt + t * _BB, _BB)])
        plsc.subcore_barrier()

        gsems = (gs0, gs1)
        ssems = (ss0, ss1)

        def gather(j, b):
            return pltpu.async_copy(y_hbm.at[gidx_v.at[j]], rows_v.at[b],
                                    gsems[b])

        def gather_wait(j, b):
            pltpu.make_async_copy(y_hbm.at[gidx_v.at[j]], rows_v.at[b],
                                  gsems[b]).wait()

        def scatter(j, b):
            return pltpu.async_copy(rows_v.at[b], acc_sh.at[omap_v.at[j]],
                                    ssems[b], add=True)

        def scatter_wait(j, b):
            pltpu.make_async_copy(rows_v.at[b], acc_sh.at[omap_v.at[j]],
                                  ssems[b]).wait()

        @pl.loop(0, nseg)
        def _seg(s):
            row0 = wid * bpt + s * _SEG
            pltpu.sync_copy(gidx_hbm.at[pl.ds(row0, _SEG)], gidx_v)
            pltpu.sync_copy(omap_hbm.at[pl.ds(row0, _SEG)], omap_v)

            gather(0, 0)

            @pl.loop(0, _SEG // 2)
            def _pairs(jj):
                j0 = jj * 2
                j1 = j0 + 1
                gather_wait(j0, 0)

                @pl.when(jj > 0)
                def _():
                    scatter_wait(j0, 1)

                gather(j1, 1)
                scatter(j0, 0)
                gather_wait(j1, 1)
                scatter(j1, 1)
                scatter_wait(j0, 0)

                @pl.when(jj < _SEG // 2 - 1)
                def _():
                    gather(j0 + 2, 0)

            scatter_wait(_SEG - 1, 1)

        plsc.subcore_barrier()
        pltpu.sync_copy(acc_sh.at[pl.ds(sid * rpt, rpt)],
                        out_hbm.at[cid, pl.ds(sid * rpt, rpt)])

    return sc_scatter


def kernel(x, imap, omap, kernel, bias):
    n, cin = x.shape
    kvol, epk = imap.shape
    cout = kernel.shape[2]

    xb = x.astype(jnp.bfloat16)
    wb = kernel.astype(jnp.bfloat16)
    bm = 2000
    y = pl.pallas_call(
        _matmul_body,
        grid=(n // bm, kvol),
        in_specs=[
            pl.BlockSpec((bm, cin), lambda i, k: (i, 0)),
            pl.BlockSpec((1, cin, cout), lambda i, k: (k, 0, 0)),
        ],
        out_specs=pl.BlockSpec((1, bm, cout), lambda i, k: (k, i, 0)),
        out_shape=jax.ShapeDtypeStruct((kvol, n, cout), jnp.float32),
    )(xb, wb)
    y = y.reshape(kvol * n, cout)

    e_total = kvol * epk
    seg_edges = _SEG * _BB
    bpt = -(-e_total // (_NW * seg_edges)) * _SEG
    e_pad = _NW * bpt * _BB
    n_acc = ((n + _NS * _BB - 1) // (_NS * _BB)) * (_NS * _BB)

    gidx = (jnp.arange(kvol, dtype=jnp.int32)[:, None] * n + imap).ravel()
    omap_f = omap.ravel()
    pad = e_pad - e_total
    pad_ids = jnp.arange(pad, dtype=jnp.int32)
    gidx = jnp.concatenate([gidx, (pad_ids * 53) % (kvol * n)])
    omap_f = jnp.concatenate([omap_f, n + pad_ids % (n_acc - n)])
    gidx = gidx.reshape(e_pad // _BB, _BB)
    omap_f = omap_f.reshape(e_pad // _BB, _BB)

    partials = _make_sc_scatter(n_acc, cout, bpt)(y, gidx, omap_f)

    br = 1000
    out = pl.pallas_call(
        _combine_body,
        grid=(n // br,),
        in_specs=[
            pl.BlockSpec((_NC, br, cout), lambda i: (0, i, 0)),
            pl.BlockSpec((1, cout), lambda i: (0, 0)),
        ],
        out_specs=pl.BlockSpec((br, cout), lambda i: (i, 0)),
        out_shape=jax.ShapeDtypeStruct((n, cout), jnp.float32),
    )(partials, bias.reshape(1, cout))
    return out


# speedup vs baseline: 1.0484x; 1.0484x over previous
import functools

import jax
import jax.numpy as jnp
from jax import lax
from jax.experimental import pallas as pl
from jax.experimental.pallas import tpu as pltpu
from jax.experimental.pallas import tpu_sc as plsc

_NC = 2
_NS = 16
_NW = _NC * _NS
_BB = 128
_SEG = 40


def _matmul_body(x_ref, w_ref, y_ref):
    y_ref[0] = jnp.dot(x_ref[...], w_ref[0], preferred_element_type=jnp.float32)


def _combine_body(p_ref, b_ref, o_ref):
    o_ref[...] = p_ref[0] + p_ref[1] + b_ref[...]


def _make_sc_scatter(n_acc, cout, bpt):
    rpt = n_acc // _NS
    nseg = bpt // _SEG

    mesh = plsc.VectorSubcoreMesh(
        core_axis_name="c", subcore_axis_name="s",
        num_cores=_NC, num_subcores=_NS)

    @functools.partial(
        pl.kernel,
        out_type=jax.ShapeDtypeStruct((_NC, n_acc, cout), jnp.float32),
        mesh=mesh,
        scratch_types=[
            pltpu.VMEM((_SEG, _BB), jnp.int32),
            pltpu.VMEM((_SEG, _BB), jnp.int32),
            pltpu.VMEM((2, _BB, cout), jnp.float32),
            pltpu.VMEM_SHARED((n_acc, cout), jnp.float32),
            pltpu.SemaphoreType.DMA,
            pltpu.SemaphoreType.DMA,
            pltpu.SemaphoreType.DMA,
            pltpu.SemaphoreType.DMA,
        ],
    )
    def sc_scatter(y_hbm, gidx_hbm, omap_hbm, out_hbm,
                   gidx_v, omap_v, rows_v, acc_sh, gs0, gs1, ss0, ss1):
        cid = lax.axis_index("c")
        sid = lax.axis_index("s")
        wid = cid * _NS + sid

        zero = jnp.zeros((16,), jnp.float32)

        @pl.loop(0, _BB)
        def _zero_rows(r):
            for q in range(cout // 16):
                rows_v[0, r, pl.ds(q * 16, 16)] = zero

        for t in range(rpt // _BB):
            pltpu.sync_copy(rows_v.at[0],
                            acc_sh.at[pl.ds(sid * rpt + t * _BB, _BB)])
        plsc.subcore_barrier()

        gsems = (gs0, gs1)
        ssems = (ss0, ss1)

        def gather(j, b):
            return pltpu.async_copy(y_hbm.at[gidx_v.at[j]], rows_v.at[b],
                                    gsems[b])

        def gather_wait(j, b):
            pltpu.make_async_copy(y_hbm.at[gidx_v.at[j]], rows_v.at[b],
                                  gsems[b]).wait()

        def scatter(j, b):
            return pltpu.async_copy(rows_v.at[b], acc_sh.at[omap_v.at[j]],
                                    ssems[b], add=True)

        def scatter_wait(j, b):
            pltpu.make_async_copy(rows_v.at[b], acc_sh.at[omap_v.at[j]],
                                  ssems[b]).wait()

        @pl.loop(0, nseg)
        def _seg(s):
            row0 = wid * bpt + s * _SEG
            pltpu.sync_copy(gidx_hbm.at[pl.ds(row0, _SEG)], gidx_v)
            pltpu.sync_copy(omap_hbm.at[pl.ds(row0, _SEG)], omap_v)

            gather(0, 0)

            @pl.loop(0, _SEG // 2)
            def _pairs(jj):
                j0 = jj * 2
                j1 = j0 + 1
                gather_wait(j0, 0)

                @pl.when(jj > 0)
                def _():
                    scatter_wait(j0, 1)

                gather(j1, 1)
                scatter(j0, 0)
                gather_wait(j1, 1)
                scatter(j1, 1)
                scatter_wait(j0, 0)

                @pl.when(jj < _SEG // 2 - 1)
                def _():
                    gather(j0 + 2, 0)

            scatter_wait(_SEG - 1, 1)

        plsc.subcore_barrier()
        pltpu.sync_copy(acc_sh.at[pl.ds(sid * rpt, rpt)],
                        out_hbm.at[cid, pl.ds(sid * rpt, rpt)])

    return sc_scatter


def kernel(x, imap, omap, kernel, bias):
    n, cin = x.shape
    kvol, epk = imap.shape
    cout = kernel.shape[2]

    bm = 2000
    y = pl.pallas_call(
        _matmul_body,
        grid=(n // bm, kvol),
        in_specs=[
            pl.BlockSpec((bm, cin), lambda i, k: (i, 0)),
            pl.BlockSpec((1, cin, cout), lambda i, k: (k, 0, 0)),
        ],
        out_specs=pl.BlockSpec((1, bm, cout), lambda i, k: (k, i, 0)),
        out_shape=jax.ShapeDtypeStruct((kvol, n, cout), jnp.float32),
    )(x, kernel)
    y = y.reshape(kvol * n, cout)

    e_total = kvol * epk
    seg_edges = _SEG * _BB
    bpt = -(-e_total // (_NW * seg_edges)) * _SEG
    e_pad = _NW * bpt * _BB
    n_acc = ((n + _NS * _BB - 1) // (_NS * _BB)) * (_NS * _BB)

    gidx = (jnp.arange(kvol, dtype=jnp.int32)[:, None] * n + imap).ravel()
    omap_f = omap.ravel()
    pad = e_pad - e_total
    pad_ids = jnp.arange(pad, dtype=jnp.int32)
    gidx = jnp.concatenate([gidx, (pad_ids * 53) % (kvol * n)])
    omap_f = jnp.concatenate([omap_f, n + pad_ids % (n_acc - n)])
    gidx = gidx.reshape(e_pad // _BB, _BB)
    omap_f = omap_f.reshape(e_pad // _BB, _BB)

    partials = _make_sc_scatter(n_acc, cout, bpt)(y, gidx, omap_f)

    br = 1000
    out = pl.pallas_call(
        _combine_body,
        grid=(n // br,),
        in_specs=[
            pl.BlockSpec((_NC, br, cout), lambda i: (0, i, 0)),
            pl.BlockSpec((1, cout), lambda i: (0, 0)),
        ],
        out_specs=pl.BlockSpec((br, cout), lambda i: (i, 0)),
        out_shape=jax.ShapeDtypeStruct((n, cout), jnp.float32),
    )(partials, bias.reshape(1, cout))
    return out


# speedup vs baseline: 1.1419x; 1.0893x over previous
import functools

import jax
import jax.numpy as jnp
from jax import lax
from jax.experimental import pallas as pl
from jax.experimental.pallas import tpu as pltpu
from jax.experimental.pallas import tpu_sc as plsc

_NC = 2
_NS = 16
_NW = _NC * _NS
_BB = 128
_SEG = 40


def _matmul_body(x_ref, w_ref, y_ref):
    y_ref[0] = jnp.dot(x_ref[...], w_ref[0], preferred_element_type=jnp.float32)


def _combine_body(p_ref, b_ref, o_ref):
    o_ref[...] = p_ref[0] + p_ref[1] + b_ref[...]


def _make_sc_scatter(n_acc, cout, bpt):
    rpt = n_acc // _NS
    nseg = bpt // _SEG

    mesh = plsc.VectorSubcoreMesh(
        core_axis_name="c", subcore_axis_name="s",
        num_cores=_NC, num_subcores=_NS)

    @functools.partial(
        pl.kernel,
        out_type=jax.ShapeDtypeStruct((_NC, n_acc, cout), jnp.float32),
        mesh=mesh,
        scratch_types=[
            pltpu.VMEM((_SEG, _BB), jnp.int32),
            pltpu.VMEM((_SEG, _BB), jnp.int32),
            pltpu.VMEM((2, _BB, cout), jnp.float32),
            pltpu.VMEM_SHARED((n_acc, cout), jnp.float32),
            pltpu.SemaphoreType.DMA,
            pltpu.SemaphoreType.DMA,
        ],
    )
    def sc_scatter(y_hbm, gidx_hbm, omap_hbm, out_hbm,
                   gidx_v, omap_v, rows_v, acc_sh, gs0, gs1):
        cid = lax.axis_index("c")
        sid = lax.axis_index("s")
        wid = cid * _NS + sid

        zero = jnp.zeros((16,), jnp.float32)

        @pl.loop(0, _BB)
        def _zero_rows(r):
            for q in range(cout // 16):
                rows_v[0, r, pl.ds(q * 16, 16)] = zero

        for t in range(rpt // _BB):
            pltpu.sync_copy(rows_v.at[0],
                            acc_sh.at[pl.ds(sid * rpt + t * _BB, _BB)])
        plsc.subcore_barrier()

        gsems = (gs0, gs1)

        def gather(j, b):
            return pltpu.async_copy(y_hbm.at[gidx_v.at[j]], rows_v.at[b],
                                    gsems[b])

        def gather_wait(j, b):
            pltpu.make_async_copy(y_hbm.at[gidx_v.at[j]], rows_v.at[b],
                                  gsems[b]).wait()

        def scatter_sync(j, b):
            pltpu.sync_copy(rows_v.at[b], acc_sh.at[omap_v.at[j]], add=True)

        @pl.loop(0, nseg)
        def _seg(s):
            row0 = wid * bpt + s * _SEG
            pltpu.sync_copy(gidx_hbm.at[pl.ds(row0, _SEG)], gidx_v)
            pltpu.sync_copy(omap_hbm.at[pl.ds(row0, _SEG)], omap_v)

            gather(0, 0)
            gather(1, 1)

            @pl.loop(0, _SEG // 2)
            def _pairs(jj):
                j0 = jj * 2
                j1 = j0 + 1
                gather_wait(j0, 0)
                scatter_sync(j0, 0)

                @pl.when(jj < _SEG // 2 - 1)
                def _():
                    gather(j0 + 2, 0)

                gather_wait(j1, 1)
                scatter_sync(j1, 1)

                @pl.when(jj < _SEG // 2 - 1)
                def _():
                    gather(j1 + 2, 1)

        plsc.subcore_barrier()
        pltpu.sync_copy(acc_sh.at[pl.ds(sid * rpt, rpt)],
                        out_hbm.at[cid, pl.ds(sid * rpt, rpt)])

    return sc_scatter


def kernel(x, imap, omap, kernel, bias):
    n, cin = x.shape
    kvol, epk = imap.shape
    cout = kernel.shape[2]

    bm = 2000
    y = pl.pallas_call(
        _matmul_body,
        grid=(n // bm, kvol),
        in_specs=[
            pl.BlockSpec((bm, cin), lambda i, k: (i, 0)),
            pl.BlockSpec((1, cin, cout), lambda i, k: (k, 0, 0)),
        ],
        out_specs=pl.BlockSpec((1, bm, cout), lambda i, k: (k, i, 0)),
        out_shape=jax.ShapeDtypeStruct((kvol, n, cout), jnp.float32),
    )(x, kernel)
    y = y.reshape(kvol * n, cout)

    e_total = kvol * epk
    seg_edges = _SEG * _BB
    bpt = -(-e_total // (_NW * seg_edges)) * _SEG
    e_pad = _NW * bpt * _BB
    n_acc = ((n + _NS * _BB - 1) // (_NS * _BB)) * (_NS * _BB)

    gidx = (jnp.arange(kvol, dtype=jnp.int32)[:, None] * n + imap).ravel()
    omap_f = omap.ravel()
    pad = e_pad - e_total
    pad_ids = jnp.arange(pad, dtype=jnp.int32)
    gidx = jnp.concatenate([gidx, (pad_ids * 53) % (kvol * n)])
    omap_f = jnp.concatenate([omap_f, n + pad_ids % (n_acc - n)])
    gidx = gidx.reshape(e_pad // _BB, _BB)
    omap_f = omap_f.reshape(e_pad // _BB, _BB)

    partials = _make_sc_scatter(n_acc, cout, bpt)(y, gidx, omap_f)

    br = 1000
    out = pl.pallas_call(
        _combine_body,
        grid=(n // br,),
        in_specs=[
            pl.BlockSpec((_NC, br, cout), lambda i: (0, i, 0)),
            pl.BlockSpec((1, cout), lambda i: (0, 0)),
        ],
        out_specs=pl.BlockSpec((br, cout), lambda i: (i, 0)),
        out_shape=jax.ShapeDtypeStruct((n, cout), jnp.float32),
    )(partials, bias.reshape(1, cout))
    return out


# speedup vs baseline: 1.2651x; 1.1079x over previous
import functools

import jax
import jax.numpy as jnp
from jax import lax
from jax.experimental import pallas as pl
from jax.experimental.pallas import tpu as pltpu
from jax.experimental.pallas import tpu_sc as plsc

_NC = 2
_NS = 16
_NW = _NC * _NS
_BB = 128
_SEG = 40


def _matmul_body(x_ref, w_ref, y_ref):
    y_ref[0] = jnp.dot(x_ref[...], w_ref[0], preferred_element_type=jnp.float32)


def _combine2_body(p_ref, q_ref, b_ref, o_ref):
    o_ref[...] = (p_ref[0] + p_ref[1]) + (q_ref[0] + q_ref[1]) + b_ref[...]


def _make_sc_scatter(n_acc, cout, bpt):
    rpt = n_acc // _NS
    nseg = bpt // _SEG

    mesh = plsc.VectorSubcoreMesh(
        core_axis_name="c", subcore_axis_name="s",
        num_cores=_NC, num_subcores=_NS)

    @functools.partial(
        pl.kernel,
        out_type=jax.ShapeDtypeStruct((_NC, n_acc, cout), jnp.float32),
        mesh=mesh,
        scratch_types=[
            pltpu.VMEM((_SEG, _BB), jnp.int32),
            pltpu.VMEM((_SEG, _BB), jnp.int32),
            pltpu.VMEM((2, _BB, cout), jnp.float32),
            pltpu.VMEM_SHARED((n_acc, cout), jnp.float32),
            pltpu.SemaphoreType.DMA,
            pltpu.SemaphoreType.DMA,
        ],
    )
    def sc_scatter(y_hbm, gidx_hbm, omap_hbm, out_hbm,
                   gidx_v, omap_v, rows_v, acc_sh, gs0, gs1):
        cid = lax.axis_index("c")
        sid = lax.axis_index("s")
        wid = cid * _NS + sid

        zero = jnp.zeros((16,), jnp.float32)

        @pl.loop(0, _BB)
        def _zero_rows(r):
            for q in range(cout // 16):
                rows_v[0, r, pl.ds(q * 16, 16)] = zero

        for t in range(rpt // _BB):
            pltpu.sync_copy(rows_v.at[0],
                            acc_sh.at[pl.ds(sid * rpt + t * _BB, _BB)])
        plsc.subcore_barrier()

        gsems = (gs0, gs1)

        def gather(j, b):
            return pltpu.async_copy(y_hbm.at[gidx_v.at[j]], rows_v.at[b],
                                    gsems[b])

        def gather_wait(j, b):
            pltpu.make_async_copy(y_hbm.at[gidx_v.at[j]], rows_v.at[b],
                                  gsems[b]).wait()

        def scatter_sync(j, b):
            pltpu.sync_copy(rows_v.at[b], acc_sh.at[omap_v.at[j]], add=True)

        @pl.loop(0, nseg)
        def _seg(s):
            row0 = wid * bpt + s * _SEG
            pltpu.sync_copy(gidx_hbm.at[pl.ds(row0, _SEG)], gidx_v)
            pltpu.sync_copy(omap_hbm.at[pl.ds(row0, _SEG)], omap_v)

            gather(0, 0)
            gather(1, 1)

            @pl.loop(0, _SEG // 2)
            def _pairs(jj):
                j0 = jj * 2
                j1 = j0 + 1
                gather_wait(j0, 0)
                scatter_sync(j0, 0)

                @pl.when(jj < _SEG // 2 - 1)
                def _():
                    gather(j0 + 2, 0)

                gather_wait(j1, 1)
                scatter_sync(j1, 1)

                @pl.when(jj < _SEG // 2 - 1)
                def _():
                    gather(j1 + 2, 1)

        plsc.subcore_barrier()
        pltpu.sync_copy(acc_sh.at[pl.ds(sid * rpt, rpt)],
                        out_hbm.at[cid, pl.ds(sid * rpt, rpt)])

    return sc_scatter


def _matmul(x, w, bm):
    n, cin = x.shape
    nk, _, cout = w.shape
    y = pl.pallas_call(
        _matmul_body,
        grid=(n // bm, nk),
        in_specs=[
            pl.BlockSpec((bm, cin), lambda i, k: (i, 0)),
            pl.BlockSpec((1, cin, cout), lambda i, k: (k, 0, 0)),
        ],
        out_specs=pl.BlockSpec((1, bm, cout), lambda i, k: (k, i, 0)),
        out_shape=jax.ShapeDtypeStruct((nk, n, cout), jnp.float32),
    )(x, w)
    return y.reshape(nk * n, cout)


def kernel(x, imap, omap, kernel, bias):
    n, cin = x.shape
    kvol, epk = imap.shape
    cout = kernel.shape[2]

    e_total = kvol * epk
    e_half = _NW * _SEG * _BB
    bpt = _SEG
    n_acc = ((n + _NS * _BB - 1) // (_NS * _BB)) * (_NS * _BB)

    k0 = -(-e_half // epk)
    k1 = e_half // epk

    gidx = (jnp.arange(kvol, dtype=jnp.int32)[:, None] * n + imap).ravel()
    omap_f = omap.ravel()
    pad = 2 * e_half - e_total
    pad_ids = jnp.arange(pad, dtype=jnp.int32)
    gidx = jnp.concatenate([gidx, k1 * n + (pad_ids * 53) % ((kvol - k1) * n)])
    omap_f = jnp.concatenate([omap_f, n + pad_ids % (n_acc - n)])
    gidx0 = gidx[:e_half].reshape(e_half // _BB, _BB)
    omap0 = omap_f[:e_half].reshape(e_half // _BB, _BB)
    gidx1 = (gidx[e_half:] - k1 * n).reshape(e_half // _BB, _BB)
    omap1 = omap_f[e_half:].reshape(e_half // _BB, _BB)

    bm = 2000
    sc = _make_sc_scatter(n_acc, cout, bpt)
    y0 = _matmul(x, kernel[:k0], bm)
    y1 = _matmul(x, kernel[k1:], bm)
    p0 = sc(y0, gidx0, omap0)
    p1 = sc(y1, gidx1, omap1)

    br = 1000
    out = pl.pallas_call(
        _combine2_body,
        grid=(n // br,),
        in_specs=[
            pl.BlockSpec((_NC, br, cout), lambda i: (0, i, 0)),
            pl.BlockSpec((_NC, br, cout), lambda i: (0, i, 0)),
            pl.BlockSpec((1, cout), lambda i: (0, 0)),
        ],
        out_specs=pl.BlockSpec((br, cout), lambda i: (i, 0)),
        out_shape=jax.ShapeDtypeStruct((n, cout), jnp.float32),
    )(p0, p1, bias.reshape(1, cout))
    return out
